# baseline (speedup 1.0000x reference)
import functools

import jax
import jax.numpy as jnp
from jax.experimental import pallas as pl
from jax.experimental.pallas import tpu as pltpu

M, D = 131072, 128
TO_ADD_CONST = 1.0
NCHUNK = 8
ROWS_PER_CHUNK = M // NCHUNK


def _body(idx_ref, x_hbm, o_hbm, row_vmem, row_sem, *sems):
    idx = idx_ref[0]
    copies = []
    for k in range(NCHUNK):
        cp = pltpu.make_async_copy(
            x_hbm.at[pl.ds(k * ROWS_PER_CHUNK, ROWS_PER_CHUNK), :],
            o_hbm.at[pl.ds(k * ROWS_PER_CHUNK, ROWS_PER_CHUNK), :],
            sems[k],
        )
        cp.start()
        copies.append(cp)

    rcp = pltpu.make_async_copy(x_hbm.at[pl.ds(idx, 1), :], row_vmem, row_sem)
    rcp.start()
    rcp.wait()
    row_vmem[...] = row_vmem[...] + TO_ADD_CONST
    for cp in copies:
        cp.wait()
    wcp = pltpu.make_async_copy(row_vmem, o_hbm.at[pl.ds(idx, 1), :], row_sem)
    wcp.start()
    wcp.wait()


@jax.jit
def _run(tensor, idx_arr):
    grid_spec = pltpu.PrefetchScalarGridSpec(
        num_scalar_prefetch=1,
        grid=(1,),
        in_specs=[pl.BlockSpec(memory_space=pl.ANY)],
        out_specs=pl.BlockSpec(memory_space=pl.ANY),
        scratch_shapes=[pltpu.VMEM((1, D), jnp.float32)]
        + [pltpu.SemaphoreType.DMA] * (NCHUNK + 1),
    )
    return pl.pallas_call(
        _body,
        grid_spec=grid_spec,
        out_shape=jax.ShapeDtypeStruct((M, D), jnp.float32),
    )(idx_arr, tensor)


def kernel(tensor, slice_index, related_index):
    idx_arr = jnp.asarray(slice_index, dtype=jnp.int32).reshape((1,))
    out = _run(tensor, idx_arr)
    return (out, slice_index, related_index)


# speedup vs baseline: 40.7710x; 40.7710x over previous
import functools

import jax
import jax.numpy as jnp
from jax.experimental import pallas as pl
from jax.experimental.pallas import tpu as pltpu

M, D = 131072, 128
TO_ADD_CONST = 1.0
BM = 4096


def _body(idx_ref, x_ref, o_ref):
    o_ref[...] = x_ref[...]
    i = pl.program_id(0)
    idx = idx_ref[0]
    base = i * BM

    @pl.when((idx >= base) & (idx < base + BM))
    def _():
        r = idx - base
        o_ref[pl.ds(r, 1), :] = x_ref[pl.ds(r, 1), :] + TO_ADD_CONST


@jax.jit
def _run(tensor, idx_arr):
    grid_spec = pltpu.PrefetchScalarGridSpec(
        num_scalar_prefetch=1,
        grid=(M // BM,),
        in_specs=[pl.BlockSpec((BM, D), lambda i, idx: (i, 0))],
        out_specs=pl.BlockSpec((BM, D), lambda i, idx: (i, 0)),
    )
    return pl.pallas_call(
        _body,
        grid_spec=grid_spec,
        out_shape=jax.ShapeDtypeStruct((M, D), jnp.float32),
    )(idx_arr, tensor)


def kernel(tensor, slice_index, related_index):
    idx_arr = jnp.asarray(slice_index, dtype=jnp.int32).reshape((1,))
    out = _run(tensor, idx_arr)
    return (out, slice_index, related_index)


# speedup vs baseline: 44.2567x; 1.0855x over previous
import functools

import jax
import jax.numpy as jnp
from jax.experimental import pallas as pl
from jax.experimental.pallas import tpu as pltpu

M, D = 131072, 128
TO_ADD_CONST = 1.0
BM = 8192


def _body(idx_ref, x_ref, o_ref):
    o_ref[...] = x_ref[...]
    i = pl.program_id(0)
    idx = idx_ref[0]
    base = i * BM

    @pl.when((idx >= base) & (idx < base + BM))
    def _():
        r = idx - base
        o_ref[pl.ds(r, 1), :] = x_ref[pl.ds(r, 1), :] + TO_ADD_CONST


@jax.jit
def _run(tensor, idx_arr):
    grid_spec = pltpu.PrefetchScalarGridSpec(
        num_scalar_prefetch=1,
        grid=(M // BM,),
        in_specs=[pl.BlockSpec((BM, D), lambda i, idx: (i, 0))],
        out_specs=pl.BlockSpec((BM, D), lambda i, idx: (i, 0)),
    )
    return pl.pallas_call(
        _body,
        grid_spec=grid_spec,
        out_shape=jax.ShapeDtypeStruct((M, D), jnp.float32),
    )(idx_arr, tensor)


def kernel(tensor, slice_index, related_index):
    idx_arr = jnp.asarray(slice_index, dtype=jnp.int32).reshape((1,))
    out = _run(tensor, idx_arr)
    return (out, slice_index, related_index)


# speedup vs baseline: 45.8199x; 1.0353x over previous
import functools

import jax
import jax.numpy as jnp
from jax.experimental import pallas as pl
from jax.experimental.pallas import tpu as pltpu

M, D = 131072, 128
TO_ADD_CONST = 1.0
BM = 16384


def _body(idx_ref, x_ref, o_ref):
    o_ref[...] = x_ref[...]
    i = pl.program_id(0)
    idx = idx_ref[0]
    base = i * BM

    @pl.when((idx >= base) & (idx < base + BM))
    def _():
        r = idx - base
        o_ref[pl.ds(r, 1), :] = x_ref[pl.ds(r, 1), :] + TO_ADD_CONST


@jax.jit
def _run(tensor, idx_arr):
    grid_spec = pltpu.PrefetchScalarGridSpec(
        num_scalar_prefetch=1,
        grid=(M // BM,),
        in_specs=[pl.BlockSpec((BM, D), lambda i, idx: (i, 0))],
        out_specs=pl.BlockSpec((BM, D), lambda i, idx: (i, 0)),
    )
    return pl.pallas_call(
        _body,
        grid_spec=grid_spec,
        out_shape=jax.ShapeDtypeStruct((M, D), jnp.float32),
    )(idx_arr, tensor)


def kernel(tensor, slice_index, related_index):
    idx_arr = jnp.asarray(slice_index, dtype=jnp.int32).reshape((1,))
    out = _run(tensor, idx_arr)
    return (out, slice_index, related_index)
